# baseline (speedup 1.0000x reference)
import functools

import jax
import jax.numpy as jnp
from jax import lax
from jax.experimental import pallas as pl
from jax.experimental.pallas import tpu as pltpu
from jax.experimental.pallas import tpu_sc as plsc

BATCH = 16384
DIM = 128
HIDDEN = 64
NUM_CORES = 2
NUM_SUBCORES = 16
NUM_WORKERS = NUM_CORES * NUM_SUBCORES
CHUNK = 128
NBUF = 4


def _sc_gather(emb_p, emb_n, idx_p, idx_n, base0, nb):
  b_per_w = nb // NUM_WORKERS
  nchunks_per_table = b_per_w // CHUNK
  nchunks = 2 * nchunks_per_table
  nbuf = min(NBUF, nchunks)
  mesh = plsc.VectorSubcoreMesh(core_axis_name="c", subcore_axis_name="s")

  @functools.partial(
      pl.kernel,
      mesh=mesh,
      out_type=[
          jax.ShapeDtypeStruct((nb, DIM), jnp.float32),
          jax.ShapeDtypeStruct((nb, DIM), jnp.float32),
      ],
      scratch_types=[
          pltpu.VMEM((b_per_w,), jnp.int32),
          pltpu.VMEM((b_per_w,), jnp.int32),
          pltpu.VMEM((nbuf, CHUNK, DIM), jnp.float32),
          pltpu.SemaphoreType.DMA((nbuf,)),
          pltpu.SemaphoreType.DMA((nbuf,)),
      ],
  )
  def gather_kernel(embp_hbm, embn_hbm, idxp_hbm, idxn_hbm,
                    outp_hbm, outn_hbm, idxp_v, idxn_v, rows_v, gsem, wsem):
    wid = lax.axis_index("s") * NUM_CORES + lax.axis_index("c")
    base = wid * b_per_w
    pltpu.sync_copy(idxp_hbm.at[pl.ds(base0 + base, b_per_w)], idxp_v)
    pltpu.sync_copy(idxn_hbm.at[pl.ds(base0 + base, b_per_w)], idxn_v)

    sched = []
    for k in range(nchunks_per_table):
      sched.append((idxp_v, embp_hbm, outp_hbm, k * CHUNK))
    for k in range(nchunks_per_table):
      sched.append((idxn_v, embn_hbm, outn_hbm, k * CHUNK))

    def g_start(c):
      idx_v, emb, _, off = sched[c]
      b = c % nbuf
      return pltpu.async_copy(
          emb.at[idx_v.at[pl.ds(off, CHUNK)]], rows_v.at[b], gsem.at[b])

    def w_start(c):
      _, _, out, off = sched[c]
      b = c % nbuf
      return pltpu.async_copy(
          rows_v.at[b], out.at[pl.ds(base + off, CHUNK)], wsem.at[b])

    gathers = [None] * nchunks
    writes = [None] * nchunks
    for c in range(nbuf):
      gathers[c] = g_start(c)
    for c in range(nchunks):
      gathers[c].wait()
      writes[c] = w_start(c)
      nxt = c + nbuf
      if nxt < nchunks:
        writes[c].wait()
        gathers[nxt] = g_start(nxt)
    for c in range(max(0, nchunks - nbuf), nchunks):
      writes[c].wait()

  return gather_kernel(emb_p, emb_n, idx_p, idx_n)


def _tc_mlp(p_rows, n_rows, w1, b1_row, w2, b2_11, nb):
  bm = min(4096, nb)
  grid = (nb // bm,)
  dn = (((1,), (1,)), ((), ()))

  def body(p_ref, n_ref, w1p_ref, w1n_ref, b1_ref, w2_ref, b2_ref, o_ref):
    h = lax.dot_general(p_ref[...], w1p_ref[...], dn,
                        preferred_element_type=jnp.float32)
    h = h + lax.dot_general(n_ref[...], w1n_ref[...], dn,
                            preferred_element_type=jnp.float32)
    h = jnp.maximum(h + b1_ref[...], 0.0)
    o_ref[...] = jnp.sum(h * w2_ref[...], axis=1, keepdims=True) + b2_ref[0, 0]

  return pl.pallas_call(
      body,
      grid=grid,
      in_specs=[
          pl.BlockSpec((bm, DIM), lambda i: (i, 0)),
          pl.BlockSpec((bm, DIM), lambda i: (i, 0)),
          pl.BlockSpec((HIDDEN, DIM), lambda i: (0, 0)),
          pl.BlockSpec((HIDDEN, DIM), lambda i: (0, 1)),
          pl.BlockSpec((1, HIDDEN), lambda i: (0, 0)),
          pl.BlockSpec((1, HIDDEN), lambda i: (0, 0)),
          pl.BlockSpec((1, 1), lambda i: (0, 0)),
      ],
      out_specs=pl.BlockSpec((bm, 1), lambda i: (i, 0)),
      out_shape=jax.ShapeDtypeStruct((nb, 1), jnp.float32),
      compiler_params=pltpu.CompilerParams(
          dimension_semantics=("parallel",)),
  )(p_rows, n_rows, w1, w1, b1_row, w2, b2_11)


@jax.jit
def kernel(x, emb_proton, emb_neutron, W1, b1, W2, b2):
  idx = x.astype(jnp.int32)
  idx_p = idx[:, 0]
  idx_n = idx[:, 1]
  b1_row = b1.reshape(1, HIDDEN)
  b2_11 = b2.reshape(1, 1)
  half = BATCH // 2
  outs = []
  for h in range(2):
    p_rows, n_rows = _sc_gather(
        emb_proton, emb_neutron, idx_p, idx_n, h * half, half)
    outs.append(_tc_mlp(p_rows, n_rows, W1, b1_row, W2, b2_11, half))
  return jnp.concatenate(outs, axis=0)


# speedup vs baseline: 1.0618x; 1.0618x over previous
import functools

import jax
import jax.numpy as jnp
from jax import lax
from jax.experimental import pallas as pl
from jax.experimental.pallas import tpu as pltpu
from jax.experimental.pallas import tpu_sc as plsc

BATCH = 16384
DIM = 128
HIDDEN = 64
NUM_CORES = 2
NUM_SUBCORES = 16
NUM_WORKERS = NUM_CORES * NUM_SUBCORES
B_PER_W = BATCH // NUM_WORKERS
CHUNK = 128
NBUF = 7
NCHUNKS_PER_TABLE = B_PER_W // CHUNK
NCHUNKS = 2 * NCHUNKS_PER_TABLE


def _sc_gather(emb_p, emb_n, idx_p, idx_n):
  mesh = plsc.VectorSubcoreMesh(core_axis_name="c", subcore_axis_name="s")

  @functools.partial(
      pl.kernel,
      mesh=mesh,
      out_type=[
          jax.ShapeDtypeStruct((BATCH, DIM), jnp.float32),
          jax.ShapeDtypeStruct((BATCH, DIM), jnp.float32),
      ],
      scratch_types=[
          pltpu.VMEM((B_PER_W,), jnp.int32),
          pltpu.VMEM((B_PER_W,), jnp.int32),
          pltpu.VMEM((NBUF, CHUNK, DIM), jnp.float32),
          pltpu.SemaphoreType.DMA((NBUF,)),
          pltpu.SemaphoreType.DMA((NBUF,)),
      ],
  )
  def gather_kernel(embp_hbm, embn_hbm, idxp_hbm, idxn_hbm,
                    outp_hbm, outn_hbm, idxp_v, idxn_v, rows_v, gsem, wsem):
    wid = lax.axis_index("s") * NUM_CORES + lax.axis_index("c")
    base = wid * B_PER_W
    pltpu.sync_copy(idxp_hbm.at[pl.ds(base, B_PER_W)], idxp_v)
    pltpu.sync_copy(idxn_hbm.at[pl.ds(base, B_PER_W)], idxn_v)

    sched = []
    for k in range(NCHUNKS_PER_TABLE):
      sched.append((idxp_v, embp_hbm, outp_hbm, k * CHUNK))
    for k in range(NCHUNKS_PER_TABLE):
      sched.append((idxn_v, embn_hbm, outn_hbm, k * CHUNK))

    def g_start(c):
      idx_v, emb, _, off = sched[c]
      b = c % NBUF
      return pltpu.async_copy(
          emb.at[idx_v.at[pl.ds(off, CHUNK)]], rows_v.at[b], gsem.at[b])

    def w_start(c):
      _, _, out, off = sched[c]
      b = c % NBUF
      return pltpu.async_copy(
          rows_v.at[b], out.at[pl.ds(base + off, CHUNK)], wsem.at[b])

    gathers = [None] * NCHUNKS
    writes = [None] * NCHUNKS
    for c in range(min(NBUF, NCHUNKS)):
      gathers[c] = g_start(c)
    for c in range(NCHUNKS):
      gathers[c].wait()
      writes[c] = w_start(c)
      nxt = c + NBUF
      if nxt < NCHUNKS:
        writes[c].wait()
        gathers[nxt] = g_start(nxt)
    for c in range(max(0, NCHUNKS - NBUF), NCHUNKS):
      writes[c].wait()

  return gather_kernel(emb_p, emb_n, idx_p, idx_n)


def _tc_mlp(p_rows, n_rows, w1, b1_row, w2, b2_11):
  bm = 8192
  grid = (BATCH // bm,)
  dn = (((1,), (1,)), ((), ()))

  def body(p_ref, n_ref, w1p_ref, w1n_ref, b1_ref, w2_ref, b2_ref, o_ref):
    h = lax.dot_general(p_ref[...], w1p_ref[...], dn,
                        preferred_element_type=jnp.float32)
    h = h + lax.dot_general(n_ref[...], w1n_ref[...], dn,
                            preferred_element_type=jnp.float32)
    h = jnp.maximum(h + b1_ref[...], 0.0)
    o_ref[...] = jnp.sum(h * w2_ref[...], axis=1, keepdims=True) + b2_ref[0, 0]

  return pl.pallas_call(
      body,
      grid=grid,
      in_specs=[
          pl.BlockSpec((bm, DIM), lambda i: (i, 0)),
          pl.BlockSpec((bm, DIM), lambda i: (i, 0)),
          pl.BlockSpec((HIDDEN, DIM), lambda i: (0, 0)),
          pl.BlockSpec((HIDDEN, DIM), lambda i: (0, 1)),
          pl.BlockSpec((1, HIDDEN), lambda i: (0, 0)),
          pl.BlockSpec((1, HIDDEN), lambda i: (0, 0)),
          pl.BlockSpec((1, 1), lambda i: (0, 0)),
      ],
      out_specs=pl.BlockSpec((bm, 1), lambda i: (i, 0)),
      out_shape=jax.ShapeDtypeStruct((BATCH, 1), jnp.float32),
      compiler_params=pltpu.CompilerParams(
          dimension_semantics=("parallel",)),
  )(p_rows, n_rows, w1, w1, b1_row, w2, b2_11)


@jax.jit
def kernel(x, emb_proton, emb_neutron, W1, b1, W2, b2):
  idx = x.astype(jnp.int32)
  idx_p = idx[:, 0]
  idx_n = idx[:, 1]
  p_rows, n_rows = _sc_gather(emb_proton, emb_neutron, idx_p, idx_n)
  b1_row = b1.reshape(1, HIDDEN)
  b2_11 = b2.reshape(1, 1)
  return _tc_mlp(p_rows, n_rows, W1, b1_row, W2, b2_11)
